# baseline (speedup 1.0000x reference)
import functools

import jax
import jax.numpy as jnp
from jax.experimental import pallas as pl
from jax.experimental.pallas import tpu as pltpu
from jax.experimental.pallas import tpu_sc as plsc

_B, _L, _C, _CURVE = 1024, 50, 3, 128
_FIN = _C * _CURVE
_DM = 128
_DLLM = 768
_E = 16
_TOPP = 2.0 / 16.0
_EPS = 1e-9
_BB = 64

_NC, _NS = 2, 16
_NW = _NC * _NS
_SPW = _B // _NW


def _logits_body(dkp_ref, gw_ref, gb_ref, out_ref):
    out_ref[...] = jnp.dot(dkp_ref[...], gw_ref[...],
                           preferred_element_type=jnp.float32) + gb_ref[...]


def _compute_logits(dkp, gate_w, gate_b2):
    return pl.pallas_call(
        _logits_body,
        out_shape=jax.ShapeDtypeStruct((_B, _E), jnp.float32),
    )(dkp, gate_w, gate_b2)


def _route_body(lg_hbm, e1_hbm, e2_hbm, w1_hbm, w2_hbm,
                lg_v, e1_v, e2_v, w1_v, w2_v):
    wid = jax.lax.axis_index("s") * _NC + jax.lax.axis_index("c")
    base = wid * _SPW
    pltpu.sync_copy(lg_hbm.at[pl.ds(base, _SPW), :], lg_v)
    lanes = jax.lax.iota(jnp.int32, 16)
    for grp in range(_SPW // 16):
        ae1 = jnp.zeros((16,), jnp.int32)
        ae2 = jnp.zeros((16,), jnp.int32)
        aw1 = jnp.zeros((16,), jnp.float32)
        aw2 = jnp.zeros((16,), jnp.float32)
        for j in range(16):
            row = lg_v[grp * 16 + j, :]
            ex = jnp.exp(row - jnp.max(row))
            probs = ex / jnp.sum(ex)
            sp = plsc.sort_key_val(probs, lanes, descending=True)
            sprobs, order = sp
            cum = plsc.cumsum(sprobs)
            msk = (cum < _TOPP) | (lanes == 0)
            gs = jnp.where(msk, sprobs, 0.0)
            ws = gs / (jnp.sum(gs) + _EPS)
            ae1 = jnp.where(lanes == j, jnp.sum(jnp.where(lanes == 0, order, 0)), ae1)
            ae2 = jnp.where(lanes == j, jnp.sum(jnp.where(lanes == 1, order, 0)), ae2)
            aw1 = jnp.where(lanes == j, jnp.sum(jnp.where(lanes == 0, ws, 0.0)), aw1)
            aw2 = jnp.where(lanes == j, jnp.sum(jnp.where(lanes == 1, ws, 0.0)), aw2)
        sl = pl.ds(grp * 16, 16)
        e1_v[sl] = ae1
        e2_v[sl] = ae2
        w1_v[sl] = aw1
        w2_v[sl] = aw2
    out_sl = pl.ds(base, _SPW)
    pltpu.sync_copy(e1_v, e1_hbm.at[out_sl])
    pltpu.sync_copy(e2_v, e2_hbm.at[out_sl])
    pltpu.sync_copy(w1_v, w1_hbm.at[out_sl])
    pltpu.sync_copy(w2_v, w2_hbm.at[out_sl])


def _compute_routing(logits):
    f = pl.kernel(
        _route_body,
        out_type=[
            jax.ShapeDtypeStruct((_B,), jnp.int32),
            jax.ShapeDtypeStruct((_B,), jnp.int32),
            jax.ShapeDtypeStruct((_B,), jnp.float32),
            jax.ShapeDtypeStruct((_B,), jnp.float32),
        ],
        mesh=plsc.VectorSubcoreMesh(core_axis_name="c", subcore_axis_name="s"),
        compiler_params=pltpu.CompilerParams(needs_layout_passes=False),
        scratch_types=[
            pltpu.VMEM((_SPW, _E), jnp.float32),
            pltpu.VMEM((_SPW,), jnp.int32),
            pltpu.VMEM((_SPW,), jnp.int32),
            pltpu.VMEM((_SPW,), jnp.float32),
            pltpu.VMEM((_SPW,), jnp.float32),
        ],
    )
    return f(logits)


def _main_body(e1_s, e2_s, w1_s, w2_s, flat_ref, ew_ref, eb_ref,
               genw_ref, genb_ref, out_ref, comb_scr, gen_scr):
    i = pl.program_id(0)
    gen_scr[...] = jax.lax.dot_general(
        flat_ref[...], genw_ref[...], (((2,), (0,)), ((), ())),
        preferred_element_type=jnp.float32)

    def body(s, carry):
        idx = i * _BB + s
        e1 = e1_s[idx]
        w1 = w1_s[idx]
        fs = flat_ref[s]
        a1 = jnp.dot(fs, ew_ref[e1], preferred_element_type=jnp.float32)
        comb_scr[...] = w1 * a1 + w1 * eb_ref[pl.ds(e1, 1), :]

        @pl.when(w2_s[idx] != 0.0)
        def _second():
            e2 = e2_s[idx]
            w2 = w2_s[idx]
            a2 = jnp.dot(fs, ew_ref[e2], preferred_element_type=jnp.float32)
            comb_scr[...] = comb_scr[...] + w2 * a2 + w2 * eb_ref[pl.ds(e2, 1), :]

        comb16 = comb_scr[...].astype(jnp.bfloat16)
        out_ref[s] = (gen_scr[s] + genb_ref[...]) + comb16.astype(jnp.float32)
        return carry

    jax.lax.fori_loop(0, _BB, body, 0)


def kernel(cycle_curve_data, DKP_embeddings, gate_W, gate_b, expert_W,
           expert_b, gen_W, gen_b):
    flat = cycle_curve_data.reshape(_B, _L, _FIN)
    gate_b2 = gate_b.reshape(1, _E)
    gen_w2 = gen_W.reshape(_FIN, _DM)
    gen_b2 = gen_b.reshape(1, _DM)

    logits = _compute_logits(DKP_embeddings, gate_W, gate_b2)
    e1, e2, w1, w2 = _compute_routing(logits)

    grid = (_B // _BB,)
    smem_spec = pl.BlockSpec(memory_space=pltpu.SMEM)
    out = pl.pallas_call(
        _main_body,
        grid=grid,
        in_specs=[
            smem_spec, smem_spec, smem_spec, smem_spec,
            pl.BlockSpec((_BB, _L, _FIN), lambda i: (i, 0, 0)),
            pl.BlockSpec((_E, _FIN, _DM), lambda i: (0, 0, 0)),
            pl.BlockSpec((_E, _DM), lambda i: (0, 0)),
            pl.BlockSpec((_FIN, _DM), lambda i: (0, 0)),
            pl.BlockSpec((1, _DM), lambda i: (0, 0)),
        ],
        out_specs=pl.BlockSpec((_BB, _L, _DM), lambda i: (i, 0, 0)),
        out_shape=jax.ShapeDtypeStruct((_B, _L, _DM), jnp.float32),
        scratch_shapes=[
            pltpu.VMEM((_L, _DM), jnp.float32),
            pltpu.VMEM((_BB, _L, _DM), jnp.float32),
        ],
    )(e1, e2, w1, w2, flat, expert_W, expert_b, gen_w2, gen_b2)
    return out


# speedup vs baseline: 1.6352x; 1.6352x over previous
import jax
import jax.numpy as jnp
from jax.experimental import pallas as pl
from jax.experimental.pallas import tpu as pltpu
from jax.experimental.pallas import tpu_sc as plsc

_B, _L, _C, _CURVE = 1024, 50, 3, 128
_FIN = _C * _CURVE
_DM = 128
_DLLM = 768
_E = 16
_TOPP = 2.0 / 16.0
_EPS = 1e-9
_BB = 32

_NC, _NS = 2, 16
_NW = _NC * _NS
_SPW = _B // _NW


def _logits_body(dkp_ref, gw_ref, gb_ref, out_ref):
    out_ref[...] = jnp.dot(dkp_ref[...], gw_ref[...],
                           preferred_element_type=jnp.float32) + gb_ref[...]


def _compute_logits(dkp, gate_w, gate_b2):
    return pl.pallas_call(
        _logits_body,
        out_shape=jax.ShapeDtypeStruct((_B, _E), jnp.float32),
    )(dkp, gate_w, gate_b2)


def _route_body(lg_hbm, e1_hbm, e2_hbm, w1_hbm, w2_hbm,
                lg_v, e1_v, e2_v, w1_v, w2_v):
    wid = jax.lax.axis_index("s") * _NC + jax.lax.axis_index("c")
    base = wid * _SPW
    pltpu.sync_copy(lg_hbm.at[pl.ds(base * _E, _SPW * _E)], lg_v)
    lanes = jax.lax.iota(jnp.int32, 16)
    for grp in range(_SPW // 16):
        ae1 = jnp.zeros((16,), jnp.int32)
        ae2 = jnp.zeros((16,), jnp.int32)
        aw1 = jnp.zeros((16,), jnp.float32)
        aw2 = jnp.zeros((16,), jnp.float32)
        for j in range(16):
            row = lg_v[pl.ds((grp * 16 + j) * _E, _E)]
            ex = jnp.exp(row - jnp.max(row))
            probs = ex / jnp.sum(ex)
            sprobs, order = plsc.sort_key_val(probs, lanes, descending=True)
            cum = plsc.cumsum(sprobs)
            msk = (cum < _TOPP) | (lanes == 0)
            gs = jnp.where(msk, sprobs, 0.0)
            ws = gs / (jnp.sum(gs) + _EPS)
            ae1 = jnp.where(lanes == j, jnp.sum(jnp.where(lanes == 0, order, 0)), ae1)
            ae2 = jnp.where(lanes == j, jnp.sum(jnp.where(lanes == 1, order, 0)), ae2)
            aw1 = jnp.where(lanes == j, jnp.sum(jnp.where(lanes == 0, ws, 0.0)), aw1)
            aw2 = jnp.where(lanes == j, jnp.sum(jnp.where(lanes == 1, ws, 0.0)), aw2)
        sl = pl.ds(grp * 16, 16)
        e1_v[sl] = ae1
        e2_v[sl] = ae2
        w1_v[sl] = aw1
        w2_v[sl] = aw2
    out_sl = pl.ds(base, _SPW)
    pltpu.sync_copy(e1_v, e1_hbm.at[out_sl])
    pltpu.sync_copy(e2_v, e2_hbm.at[out_sl])
    pltpu.sync_copy(w1_v, w1_hbm.at[out_sl])
    pltpu.sync_copy(w2_v, w2_hbm.at[out_sl])


def _compute_routing(logits_1d):
    f = pl.kernel(
        _route_body,
        out_type=[
            jax.ShapeDtypeStruct((_B,), jnp.int32),
            jax.ShapeDtypeStruct((_B,), jnp.int32),
            jax.ShapeDtypeStruct((_B,), jnp.float32),
            jax.ShapeDtypeStruct((_B,), jnp.float32),
        ],
        mesh=plsc.VectorSubcoreMesh(core_axis_name="c", subcore_axis_name="s"),
        compiler_params=pltpu.CompilerParams(needs_layout_passes=False),
        scratch_types=[
            pltpu.VMEM((_SPW * _E,), jnp.float32),
            pltpu.VMEM((_SPW,), jnp.int32),
            pltpu.VMEM((_SPW,), jnp.int32),
            pltpu.VMEM((_SPW,), jnp.float32),
            pltpu.VMEM((_SPW,), jnp.float32),
        ],
    )
    return f(logits_1d)


def _main_body(e1_s, e2_s, w1_s, w2_s, flat_ref, ew_ref, eb_ref,
               genw_ref, genb_ref, out_ref, gen_scr):
    i = pl.program_id(0)
    gen_scr[...] = jax.lax.dot_general(
        flat_ref[...], genw_ref[...], (((2,), (0,)), ((), ())),
        preferred_element_type=jnp.float32)
    genb = genb_ref[...]
    for s in range(_BB):
        idx = i * _BB + s
        e1 = e1_s[idx]
        e2 = e2_s[idx]
        w1 = w1_s[idx]
        w2 = w2_s[idx]
        fs = flat_ref[s]
        a1 = jnp.dot(fs, ew_ref[e1], preferred_element_type=jnp.float32)
        a2 = jnp.dot(fs, ew_ref[e2], preferred_element_type=jnp.float32)
        comb = (w1 * a1 + w2 * a2
                + w1 * eb_ref[pl.ds(e1, 1), :] + w2 * eb_ref[pl.ds(e2, 1), :])
        comb16 = comb.astype(jnp.bfloat16)
        out_ref[s] = (gen_scr[s] + genb) + comb16.astype(jnp.float32)


def kernel(cycle_curve_data, DKP_embeddings, gate_W, gate_b, expert_W,
           expert_b, gen_W, gen_b):
    flat = cycle_curve_data.reshape(_B, _L, _FIN).astype(jnp.bfloat16)
    gate_b2 = gate_b.reshape(1, _E)
    ew16 = expert_W.astype(jnp.bfloat16)
    gen_w2 = gen_W.reshape(_FIN, _DM).astype(jnp.bfloat16)
    gen_b2 = gen_b.reshape(1, _DM)

    logits = _compute_logits(DKP_embeddings, gate_W, gate_b2)
    e1, e2, w1, w2 = _compute_routing(logits.reshape(_B * _E))

    grid = (_B // _BB,)
    smem_spec = pl.BlockSpec(memory_space=pltpu.SMEM)
    out = pl.pallas_call(
        _main_body,
        grid=grid,
        in_specs=[
            smem_spec, smem_spec, smem_spec, smem_spec,
            pl.BlockSpec((_BB, _L, _FIN), lambda i: (i, 0, 0)),
            pl.BlockSpec((_E, _FIN, _DM), lambda i: (0, 0, 0)),
            pl.BlockSpec((_E, _DM), lambda i: (0, 0)),
            pl.BlockSpec((_FIN, _DM), lambda i: (0, 0)),
            pl.BlockSpec((1, _DM), lambda i: (0, 0)),
        ],
        out_specs=pl.BlockSpec((_BB, _L, _DM), lambda i: (i, 0, 0)),
        out_shape=jax.ShapeDtypeStruct((_B, _L, _DM), jnp.float32),
        scratch_shapes=[
            pltpu.VMEM((_BB, _L, _DM), jnp.float32),
        ],
    )(e1, e2, w1, w2, flat, ew16, expert_b, gen_w2, gen_b2)
    return out
